# baseline (speedup 1.0000x reference)
import functools

import jax
import jax.numpy as jnp
from jax import lax
from jax.experimental import pallas as pl
from jax.experimental.pallas import tpu as pltpu
from jax.experimental.pallas import tpu_sc as plsc

NC = 2
NS = 16
NW = NC * NS
L = 16

VOCAB = 1_000_000
D = 64
DP = 128
N = 16384
K = 50
KP = 56
NP = N // NW
NBUF = 4
G = NP // NBUF
R = 2
STEADY = (G - R) // R

VB = 128
NBLK = VOCAB // VB
VTAIL = VOCAB - NBLK * VB
RB = 248


@functools.partial(
    pl.kernel,
    mesh=plsc.VectorSubcoreMesh(core_axis_name="c", subcore_axis_name="s"),
    out_type=jax.ShapeDtypeStruct((VOCAB * DP,), jnp.float32),
    scratch_types=[
        pltpu.VMEM((D, VB), jnp.float32),
        pltpu.VMEM((D, VB), jnp.float32),
        pltpu.VMEM((VB * DP,), jnp.float32),
        pltpu.VMEM((VB * DP,), jnp.float32),
        pltpu.VMEM((VTAIL * D,), jnp.float32),
        pltpu.SemaphoreType.DMA,
        pltpu.SemaphoreType.DMA,
    ],
    compiler_params=pltpu.CompilerParams(use_tc_tiling_on_sc=True, needs_layout_passes=False),
)
def _transpose_kernel(
    wt_hbm, tail_hbm, out_hbm, cbuf0, cbuf1, tbuf0, tbuf1, tailv, rsem, wsem
):
    cbufs = (cbuf0, cbuf1)
    tbufs = (tbuf0, tbuf1)
    wid = lax.axis_index("s") * NC + lax.axis_index("c")
    b0 = wid * RB

    def fire_read(b, r):
        pltpu.async_copy(wt_hbm.at[:, pl.ds(b * VB, VB)], cbufs[r], rsem)

    def wait_read(r):
        pltpu.make_async_copy(
            wt_hbm.at[:, pl.ds(0, VB)], cbufs[r], rsem
        ).wait()

    def fire_write(b, r):
        pltpu.async_copy(
            tbufs[r], out_hbm.at[pl.ds(b * VB * DP, VB * DP)], wsem
        )

    def wait_write(r):
        pltpu.make_async_copy(
            tbufs[r], out_hbm.at[pl.ds(0, VB * DP)], wsem
        ).wait()

    def transpose_block(r):
        def krow(k, carry):
            for ib in range(VB // L):
                seg = cbufs[r][k, pl.ds(ib * L, L)]
                addr = (lax.iota(jnp.int32, L) + ib * L) * DP + k
                plsc.store_scatter(tbufs[r], [addr], seg)
            return carry

        lax.fori_loop(0, D, krow, 0)

    def valid(b):
        return b < NBLK

    @pl.when(valid(b0))
    def _():
        fire_read(b0, 0)

    def body(i, carry):
        for r in range(R):
            b = b0 + i * R + r
            bn = b + 1

            @pl.when(valid(b))
            def _():
                wait_read(r)

            @pl.when(valid(bn) & (bn < b0 + RB))
            def _():
                fire_read(bn, (r + 1) % R)

            @pl.when(valid(b))
            def _():
                @pl.when(i * R + r >= R)
                def _():
                    wait_write(r)
                transpose_block(r)
                fire_write(b, r)
        return carry

    lax.fori_loop(0, RB // R, body, 0)

    for r in range(R):
        @pl.when(valid(b0 + r))
        def _():
            wait_write(r)

    @pl.when(wid == NW - 1)
    def _():
        pltpu.sync_copy(tail_hbm, tailv)

        def trow(t, carry):
            for ib in range(D // L):
                seg = tailv[pl.ds(t * D + ib * L, L)]
                addr = lax.iota(jnp.int32, L) + ib * L
                plsc.store_scatter(tbufs[0], [addr + t * DP], seg)
            return carry

        lax.fori_loop(0, VTAIL, trow, 0)
        pltpu.sync_copy(
            tbufs[0].at[pl.ds(0, VTAIL * DP)],
            out_hbm.at[pl.ds(NBLK * VB * DP, VTAIL * DP)],
        )


@functools.partial(
    pl.kernel,
    mesh=plsc.VectorSubcoreMesh(core_axis_name="c", subcore_axis_name="s"),
    out_type=jax.ShapeDtypeStruct((N, KP, DP), jnp.float32),
    scratch_types=[
        pltpu.VMEM((NP, K), jnp.int32),
        pltpu.VMEM((R, NBUF, KP, DP), jnp.float32),
        pltpu.SemaphoreType.DMA,
        pltpu.SemaphoreType.DMA,
    ],
    compiler_params=pltpu.CompilerParams(use_tc_tiling_on_sc=False),
)
def _gather_kernel(table_hbm, idx_hbm, out_hbm, idx_v, bufs, gsem, wsem):
    wid = lax.axis_index("s") * NC + lax.axis_index("c")
    i0 = wid * NP

    pltpu.sync_copy(idx_hbm.at[pl.ds(i0, NP)], idx_v)

    def fire_gathers(g, r):
        for b in range(NBUF):
            pltpu.async_copy(
                table_hbm.at[idx_v.at[g * NBUF + b]],
                bufs.at[r, b, pl.ds(0, K)],
                gsem,
            )

    def wait_gathers(r):
        for b in range(NBUF):
            pltpu.make_async_copy(
                table_hbm.at[idx_v.at[0]], bufs.at[r, b, pl.ds(0, K)], gsem
            ).wait()

    def fire_write(g, r):
        pltpu.async_copy(
            bufs.at[r], out_hbm.at[pl.ds(i0 + g * NBUF, NBUF)], wsem
        )

    def wait_write(r):
        pltpu.make_async_copy(
            bufs.at[r], out_hbm.at[pl.ds(i0, NBUF)], wsem
        ).wait()

    for r in range(R):
        fire_gathers(r, r)

    def body(i, carry):
        g0 = i * R
        for r in range(R):
            wait_gathers(r)
            fire_write(g0 + r, r)
        for r in range(R):
            wait_write(r)
            fire_gathers(g0 + R + r, r)
        return carry

    lax.fori_loop(0, STEADY, body, 0)

    g0 = STEADY * R
    for r in range(R):
        wait_gathers(r)
        fire_write(g0 + r, r)
    for r in range(R):
        wait_write(r)


def kernel(plan_ids, weight):
    wt = weight.T
    tail = weight[NBLK * VB :].reshape(VTAIL * D)
    table_flat = _transpose_kernel(wt, tail)
    table_pad = table_flat.reshape(VOCAB, DP)
    out_padded = _gather_kernel(table_pad, plan_ids.astype(jnp.int32))
    return out_padded[:, :K, :D]


# speedup vs baseline: 1.6948x; 1.6948x over previous
import functools

import jax
import jax.numpy as jnp
from jax import lax
from jax.experimental import pallas as pl
from jax.experimental.pallas import tpu as pltpu
from jax.experimental.pallas import tpu_sc as plsc

NC = 2
NS = 16
NW = NC * NS

VOCAB = 1_000_000
D = 64
DP = 128
N = 16384
K = 50
KP = 56
NP = N // NW
NBUF = 4
G = NP // NBUF
R = 2
STEADY = (G - R) // R


@functools.partial(
    pl.kernel,
    mesh=plsc.VectorSubcoreMesh(core_axis_name="c", subcore_axis_name="s"),
    out_type=jax.ShapeDtypeStruct((N, KP, DP), jnp.float32),
    scratch_types=[
        pltpu.VMEM((NP, K), jnp.int32),
        pltpu.VMEM((R, NBUF, KP, DP), jnp.float32),
        pltpu.SemaphoreType.DMA,
        pltpu.SemaphoreType.DMA,
    ],
    compiler_params=pltpu.CompilerParams(use_tc_tiling_on_sc=False),
)
def _gather_kernel(table_hbm, idx_hbm, out_hbm, idx_v, bufs, gsem, wsem):
    wid = lax.axis_index("s") * NC + lax.axis_index("c")
    i0 = wid * NP

    pltpu.sync_copy(idx_hbm.at[pl.ds(i0, NP)], idx_v)

    def fire_gathers(g, r):
        for b in range(NBUF):
            pltpu.async_copy(
                table_hbm.at[idx_v.at[g * NBUF + b]],
                bufs.at[r, b, pl.ds(0, K)],
                gsem,
            )

    def wait_gathers(r):
        for b in range(NBUF):
            pltpu.make_async_copy(
                table_hbm.at[idx_v.at[0]], bufs.at[r, b, pl.ds(0, K)], gsem
            ).wait()

    def fire_write(g, r):
        pltpu.async_copy(
            bufs.at[r, :, pl.ds(0, K)],
            out_hbm.at[pl.ds(i0 + g * NBUF, NBUF), pl.ds(0, K)],
            wsem,
        )

    def wait_write(r):
        pltpu.make_async_copy(
            bufs.at[r, :, pl.ds(0, K)],
            out_hbm.at[pl.ds(i0, NBUF), pl.ds(0, K)],
            wsem,
        ).wait()

    for r in range(R):
        fire_gathers(r, r)

    def body(i, carry):
        g0 = i * R
        for r in range(R):
            wait_gathers(r)
            fire_write(g0 + r, r)
        for r in range(R):
            wait_write(r)
            fire_gathers(g0 + R + r, r)
        return carry

    lax.fori_loop(0, STEADY, body, 0)

    g0 = STEADY * R
    for r in range(R):
        wait_gathers(r)
        fire_write(g0 + r, r)
    for r in range(R):
        wait_write(r)


def kernel(plan_ids, weight):
    w2 = jnp.pad(weight, ((0, 0), (0, DP - D)))
    out_padded = _gather_kernel(w2, plan_ids.astype(jnp.int32))
    return out_padded[:, :K, :D]


# speedup vs baseline: 1.6970x; 1.0013x over previous
import functools

import jax
import jax.numpy as jnp
from jax import lax
from jax.experimental import pallas as pl
from jax.experimental.pallas import tpu as pltpu
from jax.experimental.pallas import tpu_sc as plsc

NC = 2
NS = 16
NW = NC * NS

VOCAB = 1_000_000
D = 64
DP = 128
N = 16384
K = 50
KP = 56
NP = N // NW
NBUF = 8
G = NP // NBUF
R = 2
STEADY = (G - R) // R


@functools.partial(
    pl.kernel,
    mesh=plsc.VectorSubcoreMesh(core_axis_name="c", subcore_axis_name="s"),
    out_type=jax.ShapeDtypeStruct((N, KP, DP), jnp.float32),
    scratch_types=[
        pltpu.VMEM((NP, K), jnp.int32),
        pltpu.VMEM((R, NBUF, K, DP), jnp.float32),
        pltpu.SemaphoreType.DMA,
        pltpu.SemaphoreType.DMA,
    ],
    compiler_params=pltpu.CompilerParams(use_tc_tiling_on_sc=False),
)
def _gather_kernel(table_hbm, idx_hbm, out_hbm, idx_v, bufs, gsem, wsem):
    wid = lax.axis_index("s") * NC + lax.axis_index("c")
    i0 = wid * NP

    pltpu.sync_copy(idx_hbm.at[pl.ds(i0, NP)], idx_v)

    def fire_gathers(g, r):
        for b in range(NBUF):
            pltpu.async_copy(
                table_hbm.at[idx_v.at[g * NBUF + b]],
                bufs.at[r, b],
                gsem,
            )

    def wait_gathers(r):
        for b in range(NBUF):
            pltpu.make_async_copy(
                table_hbm.at[idx_v.at[0]], bufs.at[r, b], gsem
            ).wait()

    def fire_write(g, r):
        pltpu.async_copy(
            bufs.at[r],
            out_hbm.at[pl.ds(i0 + g * NBUF, NBUF), pl.ds(0, K)],
            wsem,
        )

    def wait_write(r):
        pltpu.make_async_copy(
            bufs.at[r],
            out_hbm.at[pl.ds(i0, NBUF), pl.ds(0, K)],
            wsem,
        ).wait()

    for r in range(R):
        fire_gathers(r, r)

    def body(i, carry):
        g0 = i * R
        for r in range(R):
            wait_gathers(r)
            fire_write(g0 + r, r)
        for r in range(R):
            wait_write(r)
            fire_gathers(g0 + R + r, r)
        return carry

    lax.fori_loop(0, STEADY, body, 0)

    g0 = STEADY * R
    for r in range(R):
        wait_gathers(r)
        fire_write(g0 + r, r)
    for r in range(R):
        wait_write(r)


def kernel(plan_ids, weight):
    w2 = jnp.pad(weight, ((0, 0), (0, DP - D)))
    out_padded = _gather_kernel(w2, plan_ids.astype(jnp.int32))
    return out_padded[:, :K, :D]
